# baseline (speedup 1.0000x reference)
import functools

import jax
import jax.numpy as jnp
from jax import lax
from jax.experimental import pallas as pl
from jax.experimental.pallas import tpu as pltpu
from jax.experimental.pallas import tpu_sc as plsc

L = 16


def _make_sc_kernel(B, S, D, NB):
    info = plsc.get_sparse_core_info()
    NC, NS = info.num_cores, info.num_subcores
    NW = NC * NS
    assert B % NW == 0
    BPW = B // NW
    assert BPW % NB == 0
    ROWS = NB * S
    CH = 80
    assert ROWS % CH == 0
    NCH = ROWS // CH

    mesh = plsc.VectorSubcoreMesh(core_axis_name="c", subcore_axis_name="s")

    @functools.partial(
        pl.kernel,
        mesh=mesh,
        out_type=jax.ShapeDtypeStruct((B * S, D), jnp.float32),
        compiler_params=pltpu.CompilerParams(use_tc_tiling_on_sc=False),
        scratch_types=[
            pltpu.VMEM((ROWS,), jnp.int32),
            pltpu.VMEM((ROWS, D), jnp.float32),
            pltpu.VMEM((S, D), jnp.float32),
            pltpu.SemaphoreType.DMA,
        ],
    )
    def emb_kernel(x_ref, tab_ref, pos_ref, out_ref, idx_v, emb_v, pos_v, sem):
        wid = lax.axis_index("s") * NC + lax.axis_index("c")
        pltpu.sync_copy(pos_ref, pos_v)
        row_base = wid * BPW * S

        def group(g, carry):
            row0 = row_base + g * ROWS
            pltpu.sync_copy(x_ref.at[pl.ds(row0, ROWS)], idx_v)
            copies = [
                pltpu.async_copy(
                    tab_ref.at[idx_v.at[pl.ds(i * CH, CH)]],
                    emb_v.at[pl.ds(i * CH, CH)],
                    sem,
                )
                for i in range(NCH)
            ]
            for cp in copies:
                cp.wait()

            def add_row(r, c2):
                pr = lax.rem(r, S)
                for j in range(D // L):
                    sl = pl.ds(j * L, L)
                    emb_v[r, sl] = emb_v[r, sl] + pos_v[pr, sl]
                return c2

            lax.fori_loop(0, ROWS, add_row, 0)
            pltpu.sync_copy(emb_v, out_ref.at[pl.ds(row0, ROWS)])
            return carry

        lax.fori_loop(0, BPW // NB, group, 0)

    return emb_kernel


def kernel(x, word_table, pos_table):
    B, S = x.shape
    V, D = word_table.shape
    x_flat = x.reshape(-1).astype(jnp.int32)
    emb_fn = _make_sc_kernel(B, S, D, NB=2)
    out = emb_fn(x_flat, word_table, pos_table)
    return out.reshape(B, S, D)


# speedup vs baseline: 1.4183x; 1.4183x over previous
import functools

import jax
import jax.numpy as jnp
from jax import lax
from jax.experimental import pallas as pl
from jax.experimental.pallas import tpu as pltpu
from jax.experimental.pallas import tpu_sc as plsc

L = 16


def _make_sc_kernel(B, S, D, NB):
    info = plsc.get_sparse_core_info()
    NC, NS = info.num_cores, info.num_subcores
    NW = NC * NS
    assert B % NW == 0
    BPW = B // NW
    assert BPW % NB == 0
    ROWS = NB * S
    CH = 80
    assert ROWS % CH == 0
    NCH = ROWS // CH

    G = BPW // NB
    assert G % 2 == 0
    WROWS = BPW * S

    mesh = plsc.VectorSubcoreMesh(core_axis_name="c", subcore_axis_name="s")

    @functools.partial(
        pl.kernel,
        mesh=mesh,
        out_type=jax.ShapeDtypeStruct((B * S, D), jnp.float32),
        compiler_params=pltpu.CompilerParams(use_tc_tiling_on_sc=False),
        scratch_types=[
            pltpu.VMEM((WROWS,), jnp.int32),
            pltpu.VMEM((ROWS, D), jnp.float32),
            pltpu.VMEM((ROWS, D), jnp.float32),
            pltpu.VMEM((S, D), jnp.float32),
            pltpu.SemaphoreType.DMA,
            pltpu.SemaphoreType.DMA,
            pltpu.SemaphoreType.DMA,
            pltpu.SemaphoreType.DMA,
        ],
    )
    def emb_kernel(x_ref, tab_ref, pos_ref, out_ref,
                   idx_v, emb_a, emb_b, pos_v, sga, sgb, swa, swb):
        wid = lax.axis_index("s") * NC + lax.axis_index("c")
        row_base = wid * WROWS
        pltpu.sync_copy(x_ref.at[pl.ds(row_base, WROWS)], idx_v)
        pltpu.sync_copy(pos_ref, pos_v)

        def fire_gather(g, buf, sem):
            return [
                pltpu.async_copy(
                    tab_ref.at[idx_v.at[pl.ds(g * ROWS + i * CH, CH)]],
                    buf.at[pl.ds(i * CH, CH)],
                    sem,
                )
                for i in range(NCH)
            ]

        def wait_gather(buf, sem):
            for i in range(NCH):
                pltpu.make_async_copy(
                    tab_ref.at[idx_v.at[pl.ds(i * CH, CH)]],
                    buf.at[pl.ds(i * CH, CH)],
                    sem,
                ).wait()

        def fire_writeout(g, buf, sem):
            return pltpu.async_copy(
                buf, out_ref.at[pl.ds(row_base + g * ROWS, ROWS)], sem)

        def wait_writeout(buf, sem):
            pltpu.make_async_copy(
                buf, out_ref.at[pl.ds(row_base, ROWS)], sem).wait()

        def add_pos(buf):
            def add_s(s, c2):
                for j in range(NB):
                    r = j * S + s
                    for cchunk in range(D // L):
                        sl = pl.ds(cchunk * L, L)
                        buf[r, sl] = buf[r, sl] + pos_v[s, sl]
                return c2
            lax.fori_loop(0, S, add_s, 0)

        fire_gather(0, emb_a, sga)

        def body(h, carry):
            ga = 2 * h
            gb = 2 * h + 1
            @pl.when(h > 0)
            def _():
                wait_writeout(emb_b, swb)
            fire_gather(gb, emb_b, sgb)
            wait_gather(emb_a, sga)
            add_pos(emb_a)
            fire_writeout(ga, emb_a, swa)
            @pl.when(h + 1 < G // 2)
            def _():
                wait_writeout(emb_a, swa)
                fire_gather(ga + 2, emb_a, sga)
            wait_gather(emb_b, sgb)
            add_pos(emb_b)
            fire_writeout(gb, emb_b, swb)
            return carry

        lax.fori_loop(0, G // 2, body, 0)
        wait_writeout(emb_a, swa)
        wait_writeout(emb_b, swb)

    return emb_kernel


def kernel(x, word_table, pos_table):
    B, S = x.shape
    V, D = word_table.shape
    x_flat = x.reshape(-1).astype(jnp.int32)
    emb_fn = _make_sc_kernel(B, S, D, NB=2)
    out = emb_fn(x_flat, word_table, pos_table)
    return out.reshape(B, S, D)
